# Initial kernel scaffold; baseline (speedup 1.0000x reference)
import jax
import jax.numpy as jnp
from jax.experimental import pallas as pl


def kernel(node_type, origin_node_depth, node_depth, edge_index, edge_attr, batch, type_emb, depth_emb, W0, b0, We0, p0, W1, b1, We1, p1, W2, b2, We2, p2, Wp, bp):
    raise NotImplementedError("write your pallas kernel here")


# speedup vs baseline: 7.5901x; 7.5901x over previous
import functools

import jax
import jax.numpy as jnp
from jax import lax
from jax.experimental import pallas as pl
from jax.experimental.pallas import tpu as pltpu
from jax.experimental.pallas import tpu_sc as plsc

N = 10000
NP = 10240
E = 320000
EP = 327680
D = 128
DE = 16
NG = 256
MD = 20
NT = 100
ROWS = NP // 128
EPT = EP // 32
ECH = EPT // 128
RPT = NP // 32
DRT = NP // 16

f32 = jnp.float32
i32 = jnp.int32


def _split7(v):
    return lax.shift_right_logical(v, 7), lax.bitwise_and(v, 127)


def _split4(v):
    return lax.shift_right_logical(v, 4), lax.bitwise_and(v, 15)

_mesh = plsc.VectorSubcoreMesh(core_axis_name="c", subcore_axis_name="s")
_sc_params = pltpu.CompilerParams(needs_layout_passes=False)


def _rowmul(rows2d, vec2d):
    r = vec2d.shape[0]
    b = lax.broadcast_in_dim(vec2d, (r, 128, 128), (0, 1))
    return (rows2d.reshape(r, 128, 128) * b).reshape(r * 128, 128)


def _sds(shape, dtype):
    return jax.ShapeDtypeStruct(shape, dtype)


def _sc_embed_deg_body(temb, demb, ntype, odep, validf, srcr, dstr, znp2,
                       h0a, h0b, degp,
                       idx_a, idx_b, rows_a, rows_b, vbuf, degbuf, sbuf, dbuf):
    c = lax.axis_index("c")
    s = lax.axis_index("s")
    wid = c * 16 + s
    for k in range(4):
        base = wid * RPT + k * 80
        pltpu.sync_copy(ntype.at[pl.ds(base, 80)], idx_a)
        pltpu.sync_copy(odep.at[pl.ds(base, 80)], idx_b)
        for j in range(5):
            v = idx_b[pl.ds(16 * j, 16)]
            idx_b[pl.ds(16 * j, 16)] = jnp.minimum(jnp.maximum(v, 0), MD)
        pltpu.sync_copy(temb.at[idx_a], rows_a)
        pltpu.sync_copy(demb.at[idx_b], rows_b)
        pltpu.sync_copy(rows_a, h0a.at[pl.ds(base, 80)])
        pltpu.sync_copy(rows_b, h0b.at[pl.ds(base, 80)])
    pltpu.sync_copy(validf, vbuf)
    pltpu.sync_copy(znp2, degbuf)

    def degstep(k, carry):
        base = wid * EPT + k * 128
        pltpu.sync_copy(srcr.at[pl.ds(base, 128)], sbuf)
        pltpu.sync_copy(dstr.at[pl.ds(base, 128)], dbuf)
        for j in range(8):
            sv = sbuf[pl.ds(16 * j, 16)]
            dv = dbuf[pl.ds(16 * j, 16)]
            sr, sc = _split7(sv)
            dr, dc = _split7(dv)
            vs = plsc.load_gather(vbuf, [sr, sc])
            plsc.addupdate_scatter(degbuf, [dr, dc], vs)
        return carry

    lax.fori_loop(0, ECH, degstep, 0)
    pltpu.sync_copy(degbuf, degp.at[wid])


_sc_embed_deg = pl.kernel(
    _sc_embed_deg_body,
    out_type=[_sds((NP, D), f32), _sds((NP, D), f32), _sds((32, ROWS, 128), f32)],
    mesh=_mesh,
    compiler_params=_sc_params,
    scratch_types=[
        pltpu.VMEM((80,), i32), pltpu.VMEM((80,), i32),
        pltpu.VMEM((80, D), f32), pltpu.VMEM((80, D), f32),
        pltpu.VMEM((ROWS, 128), f32), pltpu.VMEM((ROWS, 128), f32),
        pltpu.VMEM((128,), i32), pltpu.VMEM((128,), i32),
    ],
)


def _sc_edge_body(gtab, disv, srcr, dstr, ea2, z128,
                  accp,
                  dvbuf, sbuf, dbuf, dsbuf, rows, eabuf, acc_sh):
    c = lax.axis_index("c")
    s = lax.axis_index("s")
    for m in range(5):
        r0 = s * DRT + m * 128
        pltpu.sync_copy(z128, acc_sh.at[pl.ds(r0, 128)])
    pltpu.sync_copy(disv, dvbuf)
    plsc.subcore_barrier()

    riota = lax.iota(i32, 16)
    coff = c * NP

    def step(k, carry):
        base = s * 20480 + k * 128
        pltpu.sync_copy(srcr.at[pl.ds(base, 128)], sbuf)
        pltpu.sync_copy(dstr.at[pl.ds(base, 128)], dbuf)
        ea_mine = ((k < 80) & (c == 0)) | ((k >= 80) & (c == 1))

        @pl.when(ea_mine)
        def _():
            for j in range(8):
                sv = sbuf[pl.ds(16 * j, 16)]
                sr, scc = _split7(sv)
                dsv = plsc.load_gather(dvbuf, [sr, scc])
                dsbuf[pl.ds(16 * j, 16)] = dsv

        for j in range(8):
            sbuf[pl.ds(16 * j, 16)] = sbuf[pl.ds(16 * j, 16)] + coff
        pltpu.sync_copy(gtab.at[sbuf], rows)

        @pl.when(ea_mine)
        def _():
            pltpu.sync_copy(ea2.at[pl.ds(s * 2560 + k * 16, 16)], eabuf)
            for j in range(8):
                dv16 = dsbuf[pl.ds(16 * j, 16)]
                rowv = riota + (16 * j)
                for col in range(16):
                    colv = jnp.full((16,), 64 + col, i32)
                    plsc.store_scatter(rows, [rowv, colv], dv16)
            for m in range(128):
                rows[m, pl.ds(64, 16)] = (rows[m, pl.ds(64, 16)]
                                          * eabuf[m // 8, pl.ds((m % 8) * 16, 16)])

        pltpu.sync_copy(rows, acc_sh.at[dbuf], add=True)
        return carry

    lax.fori_loop(0, EP // 16 // 128, step, 0)
    plsc.subcore_barrier()
    for m in range(5):
        r0 = s * DRT + m * 128
        pltpu.sync_copy(acc_sh.at[pl.ds(r0, 128)], accp.at[pl.ds(c * NP + r0, 128)])


_sc_edge = pl.kernel(
    _sc_edge_body,
    out_type=[_sds((2 * NP, D), f32)],
    mesh=_mesh,
    compiler_params=_sc_params,
    scratch_types=[
        pltpu.VMEM((ROWS, 128), f32),
        pltpu.VMEM((128,), i32), pltpu.VMEM((128,), i32),
        pltpu.VMEM((128,), f32),
        pltpu.VMEM((128, D), f32),
        pltpu.VMEM((16, 128), f32),
        pltpu.VMEM_SHARED((NP, D), f32),
    ],
)


def _sc_deg_pool_body(vf_next, vf_cur, srcr, dstr, hn, batch, znp2, zng, zrows,
                      degp, poolp, cntp,
                      vbuf, degbuf, sbuf, dbuf, bbuf, vcbuf, hrows, cntbuf, pool_sh):
    c = lax.axis_index("c")
    s = lax.axis_index("s")
    wid = c * 16 + s
    pltpu.sync_copy(zrows, pool_sh.at[pl.ds(16 * s, 16)])
    pltpu.sync_copy(zng, cntbuf)
    pltpu.sync_copy(vf_next, vbuf)
    pltpu.sync_copy(znp2, degbuf)
    plsc.subcore_barrier()

    def degstep(k, carry):
        base = wid * EPT + k * 128
        pltpu.sync_copy(srcr.at[pl.ds(base, 128)], sbuf)
        pltpu.sync_copy(dstr.at[pl.ds(base, 128)], dbuf)
        for j in range(8):
            sv = sbuf[pl.ds(16 * j, 16)]
            dv = dbuf[pl.ds(16 * j, 16)]
            sr, sc = _split7(sv)
            dr, dc = _split7(dv)
            vs = plsc.load_gather(vbuf, [sr, sc])
            plsc.addupdate_scatter(degbuf, [dr, dc], vs)
        return carry

    lax.fori_loop(0, ECH, degstep, 0)
    pltpu.sync_copy(degbuf, degp.at[wid])
    for k in range(4):
        base = wid * RPT + k * 80
        pltpu.sync_copy(batch.at[pl.ds(base, 80)], bbuf)
        pltpu.sync_copy(hn.at[pl.ds(base, 80)], hrows)
        pltpu.sync_copy(hrows, pool_sh.at[bbuf], add=True)
        pltpu.sync_copy(vf_cur.at[pl.ds(base, 80)], vcbuf)
        for j in range(5):
            bv = bbuf[pl.ds(16 * j, 16)]
            vv = vcbuf[pl.ds(16 * j, 16)]
            br, bc = _split7(bv)
            plsc.addupdate_scatter(cntbuf, [br, bc], vv)
    plsc.subcore_barrier()
    pltpu.sync_copy(pool_sh.at[pl.ds(16 * s, 16)],
                    poolp.at[pl.ds(c * NG + 16 * s, 16)])
    pltpu.sync_copy(cntbuf, cntp.at[wid])


_sc_deg_pool = pl.kernel(
    _sc_deg_pool_body,
    out_type=[_sds((32, ROWS, 128), f32), _sds((2 * NG, D), f32),
              _sds((32, 2, 128), f32)],
    mesh=_mesh,
    compiler_params=_sc_params,
    scratch_types=[
        pltpu.VMEM((ROWS, 128), f32), pltpu.VMEM((ROWS, 128), f32),
        pltpu.VMEM((128,), i32), pltpu.VMEM((128,), i32),
        pltpu.VMEM((80,), i32), pltpu.VMEM((80,), f32),
        pltpu.VMEM((80, D), f32),
        pltpu.VMEM((2, 128), f32),
        pltpu.VMEM_SHARED((NG, D), f32),
    ],
)


def _sc_pool_fin_body(hn, h3, vf2, vf3, batch, zng, zrows,
                      poolp2, cntp2, poolp3, cntp3,
                      bbuf, v2buf, v3buf, hrows2, hrows3, cnt2, cnt3,
                      pool2_sh, pool3_sh):
    c = lax.axis_index("c")
    s = lax.axis_index("s")
    wid = c * 16 + s
    pltpu.sync_copy(zrows, pool2_sh.at[pl.ds(16 * s, 16)])
    pltpu.sync_copy(zrows, pool3_sh.at[pl.ds(16 * s, 16)])
    pltpu.sync_copy(zng, cnt2)
    pltpu.sync_copy(zng, cnt3)
    plsc.subcore_barrier()
    for k in range(4):
        base = wid * RPT + k * 80
        pltpu.sync_copy(batch.at[pl.ds(base, 80)], bbuf)
        pltpu.sync_copy(hn.at[pl.ds(base, 80)], hrows2)
        pltpu.sync_copy(h3.at[pl.ds(base, 80)], hrows3)
        pltpu.sync_copy(hrows2, pool2_sh.at[bbuf], add=True)
        pltpu.sync_copy(hrows3, pool3_sh.at[bbuf], add=True)
        pltpu.sync_copy(vf2.at[pl.ds(base, 80)], v2buf)
        pltpu.sync_copy(vf3.at[pl.ds(base, 80)], v3buf)
        for j in range(5):
            bv = bbuf[pl.ds(16 * j, 16)]
            br, bc = _split7(bv)
            plsc.addupdate_scatter(cnt2, [br, bc], v2buf[pl.ds(16 * j, 16)])
            plsc.addupdate_scatter(cnt3, [br, bc], v3buf[pl.ds(16 * j, 16)])
    plsc.subcore_barrier()
    pltpu.sync_copy(pool2_sh.at[pl.ds(16 * s, 16)],
                    poolp2.at[pl.ds(c * NG + 16 * s, 16)])
    pltpu.sync_copy(pool3_sh.at[pl.ds(16 * s, 16)],
                    poolp3.at[pl.ds(c * NG + 16 * s, 16)])
    pltpu.sync_copy(cnt2, cntp2.at[wid])
    pltpu.sync_copy(cnt3, cntp3.at[wid])


_sc_pool_fin = pl.kernel(
    _sc_pool_fin_body,
    out_type=[_sds((2 * NG, D), f32), _sds((32, 2, 128), f32),
              _sds((2 * NG, D), f32), _sds((32, 2, 128), f32)],
    mesh=_mesh,
    compiler_params=_sc_params,
    scratch_types=[
        pltpu.VMEM((80,), i32), pltpu.VMEM((80,), f32), pltpu.VMEM((80,), f32),
        pltpu.VMEM((80, D), f32), pltpu.VMEM((80, D), f32),
        pltpu.VMEM((2, 128), f32), pltpu.VMEM((2, 128), f32),
        pltpu.VMEM_SHARED((NG, D), f32), pltpu.VMEM_SHARED((NG, D), f32),
    ],
)


def _tc1_one_body(degp, hc, fact, vf, disv_ref, g3_ref):
    deg = jnp.sum(degp[...], axis=0) + 1.0
    disv = (1.0 / jnp.sqrt(deg)) * vf[...]
    disv_ref[...] = disv
    g = _rowmul(hc[...], fact[...] * disv)
    z = jnp.zeros((1024, 64), f32)
    g3_ref[0] = jnp.concatenate([g[:, :64], z], axis=1)
    g3_ref[1] = jnp.concatenate([g[:, 64:], z], axis=1)


def _tc1_two_body(degp, hc, hc2, fact, vf, disv_ref, g3_ref):
    deg = jnp.sum(degp[...], axis=0) + 1.0
    disv = (1.0 / jnp.sqrt(deg)) * vf[...]
    disv_ref[...] = disv
    g = _rowmul(hc[...] + hc2[...], fact[...] * disv)
    z = jnp.zeros((1024, 64), f32)
    g3_ref[0] = jnp.concatenate([g[:, :64], z], axis=1)
    g3_ref[1] = jnp.concatenate([g[:, 64:], z], axis=1)


def _make_tc1(two):
    body = _tc1_two_body if two else _tc1_one_body
    spec_rows = pl.BlockSpec((1024, D), lambda i: (i, 0))
    spec_vec = pl.BlockSpec((8, 128), lambda i: (i, 0))
    spec_g3 = pl.BlockSpec((2, 1024, D), lambda i: (0, i, 0))
    in_specs = [pl.BlockSpec((32, 8, 128), lambda i: (0, i, 0)), spec_rows]
    if two:
        in_specs.append(spec_rows)
    in_specs += [spec_vec, spec_vec]
    return pl.pallas_call(
        body,
        grid=(10,),
        in_specs=in_specs,
        out_specs=[spec_vec, spec_g3],
        out_shape=[_sds((ROWS, 128), f32), _sds((2, NP, D), f32)],
    )


_tc1_one = _make_tc1(False)
_tc1_two = _make_tc1(True)


def _tc2a_body(acc0, acc1, g3, disv, vf, We, W, b, p, pb, hn_ref, sc_ref):
    a0 = acc0[...]
    a1 = acc1[...]
    eat = a0[:, 64:80] + a1[:, 64:80]
    g3v = g3[...]
    g = jnp.concatenate([g3v[0][:, :64], g3v[1][:, :64]], axis=1)
    acc = jnp.concatenate([a0[:, :64], a1[:, :64]], axis=1)
    agg = acc + g + jnp.dot(eat, We[...], preferred_element_type=f32,
                            precision=lax.Precision.HIGHEST)
    agg = _rowmul(agg, disv[...])
    hn = jnp.dot(agg.astype(jnp.bfloat16), W[...],
                 preferred_element_type=f32) + b[...]
    hn = _rowmul(jnp.maximum(hn, 0.0), vf[...])
    hn_ref[...] = hn
    pv = p[...]
    pn = jnp.sqrt(jnp.sum(pv * pv)) + 1e-12
    hb = hn.astype(jnp.bfloat16).astype(f32)
    p3 = lax.broadcast_in_dim(pb[...].reshape(D), (8, 128, D), (2,))
    sc = jnp.sum(hb.reshape(8, 128, D) * p3, axis=2) / pn
    sc_ref[...] = jnp.where(vf[...] > 0.0, sc, -jnp.inf)


_spec_rows = pl.BlockSpec((1024, D), lambda i: (i, 0))
_spec_ea = pl.BlockSpec((1024, DE), lambda i: (i, 0))
_spec_vec = pl.BlockSpec((8, 128), lambda i: (i, 0))
_spec_full = lambda shape: pl.BlockSpec(shape, lambda i: tuple(0 for _ in shape))

_spec_g3 = pl.BlockSpec((2, 1024, D), lambda i: (0, i, 0))

_tc2a = pl.pallas_call(
    _tc2a_body,
    grid=(10,),
    in_specs=[_spec_rows, _spec_rows, _spec_g3,
              _spec_vec, _spec_vec,
              _spec_full((DE, D)), _spec_full((D, D)),
              _spec_full((1, D)), _spec_full((1, D)), _spec_full((1, D))],
    out_specs=[_spec_rows, _spec_vec],
    out_shape=[_sds((NP, D), f32), _sds((ROWS, 128), f32)],
)


def _tc2b_body(k_sel, emit_h3, *refs):
    if emit_h3:
        sc_ref, vf_ref, hn_ref, fact_ref, vfn_ref, h3_ref = refs
    else:
        sc_ref, vf_ref, fact_ref, vfn_ref = refs
    scv = sc_ref[...]
    ki = lax.bitcast_convert_type(scv, i32)
    keys = lax.bitcast_convert_type(
        jnp.where(ki < 0, ~ki, ki | jnp.int32(-2147483648)), jnp.uint32)
    thr = jnp.uint32(0)
    for bpos in range(31, -1, -1):
        cand = thr | jnp.uint32(1 << bpos)
        cnt = jnp.sum((keys >= cand).astype(i32))
        thr = jnp.where(cnt >= k_sel, cand, thr)
    sel = (keys >= thr).astype(f32)
    fact = jnp.tanh(scv) * sel
    fact_ref[...] = fact
    vfn_ref[...] = vf_ref[...] * sel
    if emit_h3:
        h3_ref[...] = _rowmul(hn_ref[...], fact)


def _make_tc2b(k_sel, emit_h3):
    ins = [_sds((ROWS, 128), f32), _sds((ROWS, 128), f32)]
    outs = [_sds((ROWS, 128), f32), _sds((ROWS, 128), f32)]
    if emit_h3:
        ins.append(_sds((NP, D), f32))
        outs.append(_sds((NP, D), f32))
    return pl.pallas_call(
        functools.partial(_tc2b_body, k_sel, emit_h3),
        out_shape=outs,
    )


_tc2b_l0 = _make_tc2b(5000, False)
_tc2b_l1 = _make_tc2b(2500, False)
_tc2b_l2 = _make_tc2b(1250, True)


def _tc_fin_body(p0, p1, p2, p3, c0, c1, c2, c3, Wpp, bpp, out_ref):
    rep = jnp.zeros((NG, D), f32)
    for pp, cc in ((p0, c0), (p1, c1), (p2, c2), (p3, c3)):
        inv = 1.0 / jnp.maximum(jnp.sum(cc[...], axis=0), 1.0)
        ptot = pp[...]
        rep = rep + _rowmul(ptot[:NG] + ptot[NG:], inv)
    out_ref[...] = jnp.dot(rep.astype(jnp.bfloat16), Wpp[...],
                            preferred_element_type=f32) + bpp[...]


_tc_fin = pl.pallas_call(
    _tc_fin_body,
    out_shape=[_sds((NG, D), f32)],
)


def kernel(node_type, origin_node_depth, node_depth, edge_index, edge_attr,
           batch, type_emb, depth_emb, W0, b0, We0, p0, W1, b1, We1, p1,
           W2, b2, We2, p2, Wp, bp):
    ei = edge_index.astype(i32)
    srcp = jnp.concatenate([ei[0], jnp.full((EP - E,), N, i32)])
    dstp = jnp.concatenate([ei[1], jnp.full((EP - E,), N, i32)])
    ea_t = lax.reduce_precision(edge_attr.astype(f32), 8, 7)
    eap = jnp.concatenate([ea_t, jnp.zeros((EP - E, DE), f32)])
    ea2 = eap.reshape(EP * DE // 128, 128)
    ntp = jnp.concatenate([node_type.astype(i32), jnp.zeros((NP - N,), i32)])
    odp = jnp.concatenate([origin_node_depth.astype(i32), jnp.zeros((NP - N,), i32)])
    batchp = jnp.concatenate([batch.astype(i32), jnp.zeros((NP - N,), i32)])
    vf0 = jnp.concatenate([jnp.ones((N,), f32), jnp.zeros((NP - N,), f32)])
    znp2 = jnp.zeros((ROWS, 128), f32)
    zng = jnp.zeros((2, 128), f32)
    z128 = jnp.zeros((128, D), f32)
    zrows = jnp.zeros((16, D), f32)
    Wpp = jnp.zeros((D, D), f32).at[:, :1].set(Wp.astype(f32)).astype(jnp.bfloat16)
    bpp = jnp.zeros((1, D), f32).at[0, :1].set(bp.astype(f32))

    h0a, h0b, degp = _sc_embed_deg(type_emb.astype(f32), depth_emb.astype(f32),
                                   ntp, odp, vf0.reshape(ROWS, 128),
                                   srcp, dstp, znp2)

    layer_w = ((W0, b0, We0, p0), (W1, b1, We1, p1), (W2, b2, We2, p2))
    tc2bs = (_tc2b_l0, _tc2b_l1, _tc2b_l2)
    fact2 = jnp.ones((ROWS, 128), f32)
    vf2 = vf0.reshape(ROWS, 128)
    hcur = h0a
    hcur2 = h0b
    pools = []
    cnts = []
    for l in range(3):
        W, b, We, p = layer_w[l]
        if l == 0:
            disv2, g3 = _tc1_two(degp, hcur, hcur2, fact2, vf2)
        else:
            disv2, g3 = _tc1_one(degp, hcur, fact2, vf2)
        (accp,) = _sc_edge(g3.reshape(2 * NP, D), disv2, srcp, dstp, ea2, z128)
        pf = p.astype(f32).reshape(1, D)
        hn, sc2 = _tc2a(accp[:NP], accp[NP:], g3, disv2, vf2,
                        lax.reduce_precision(We.astype(f32), 8, 7),
                        W.astype(f32).astype(jnp.bfloat16),
                        b.astype(f32).reshape(1, D), pf,
                        lax.reduce_precision(pf, 8, 7))
        if l < 2:
            fact2, vf2n = tc2bs[l](sc2, vf2)
            degp, poolp, cntp = _sc_deg_pool(vf2n, vf2.reshape(NP),
                                             srcp, dstp, hn, batchp,
                                             znp2, zng, zrows)
            pools.append(poolp)
            cnts.append(cntp)
        else:
            fact2, vf2n, h3 = tc2bs[l](sc2, vf2, hn)
            poolp2, cntp2, poolp3, cntp3 = _sc_pool_fin(
                hn, h3, vf2.reshape(NP), vf2n.reshape(NP), batchp, zng, zrows)
            pools += [poolp2, poolp3]
            cnts += [cntp2, cntp3]
        vf2 = vf2n
        hcur = hn
    (predf,) = _tc_fin(pools[0], pools[1], pools[2], pools[3],
                       cnts[0], cnts[1], cnts[2], cnts[3], Wpp, bpp)
    pred = predf[:, :1]
    return (pred, jnp.zeros((), f32))
